# Initial kernel scaffold; baseline (speedup 1.0000x reference)
import jax
import jax.numpy as jnp
from jax.experimental import pallas as pl


def kernel(user_ids, item_ids, hist_ids, user_table, item_table, hist_table, user_bias, item_bias):
    raise NotImplementedError("write your pallas kernel here")


# speedup vs baseline: 1.2696x; 1.2696x over previous
import functools

import jax
import jax.numpy as jnp
from jax import lax
from jax.experimental import pallas as pl
from jax.experimental.pallas import tpu as pltpu
from jax.experimental.pallas import tpu_sc as plsc

_B, _L, _D = 16384, 50, 32
_AVG = 3.5
_NC, _NS = 2, 16
_NW = _NC * _NS
_PER_W = _B // _NW
_CHUNK = 16
_NCHUNK = _PER_W // _CHUNK
_ROWS = _CHUNK * _L
_INV_SQRT_L = float(1.0 / (_L ** 0.5))


def _body(uid_hbm, iid_hbm, hid_hbm, ut_hbm, it_hbm, ht_hbm, bias_hbm,
          out_hbm,
          uid_v, iid_v, urow_v, irow_v, hid_v, hrow_v, bias_v, out_v,
          stage_v, sem_u, sem_i, sem_h0, sem_h1):
    wid = lax.axis_index("s") * _NC + lax.axis_index("c")
    base = wid * _PER_W
    hbase = base * _L
    sems = (sem_h0, sem_h1)

    pltpu.sync_copy(bias_hbm, bias_v)
    bias_vec = bias_v[...]

    pltpu.sync_copy(uid_hbm.at[pl.ds(base, _PER_W)], uid_v)
    pltpu.sync_copy(iid_hbm.at[pl.ds(base, _PER_W)], iid_v)
    cp_u = pltpu.async_copy(ut_hbm.at[uid_v], urow_v, sem_u)
    cp_i = pltpu.async_copy(it_hbm.at[iid_v], irow_v, sem_i)

    def start(c, b):
        pltpu.sync_copy(hid_hbm.at[pl.ds(hbase + c * _ROWS, _ROWS)],
                        hid_v.at[b])
        pltpu.async_copy(ht_hbm.at[hid_v.at[b]], hrow_v.at[b], sems[b])

    start(0, 0)
    start(1, 1)
    cp_u.wait()
    cp_i.wait()

    def outer(o, carry):
        for b in range(2):
            c = 2 * o + b
            pltpu.make_async_copy(ht_hbm.at[hid_v.at[b]], hrow_v.at[b],
                                  sems[b]).wait()

            lane = lax.iota(jnp.int32, 16)

            def ex_body(e, carry, b=b, c=c):
                r0 = e * _L
                a0 = jnp.zeros((16,), jnp.float32)
                a1 = jnp.zeros((16,), jnp.float32)
                a2 = jnp.zeros((16,), jnp.float32)
                a3 = jnp.zeros((16,), jnp.float32)
                for l in range(0, _L, 2):
                    a0 = a0 + hrow_v[b, r0 + l, pl.ds(0, 16)]
                    a1 = a1 + hrow_v[b, r0 + l, pl.ds(16, 16)]
                    a2 = a2 + hrow_v[b, r0 + l + 1, pl.ds(0, 16)]
                    a3 = a3 + hrow_v[b, r0 + l + 1, pl.ds(16, 16)]
                h0 = (a0 + a2) * _INV_SQRT_L
                h1 = (a1 + a3) * _INV_SQRT_L
                g = c * _CHUNK + e
                u0 = urow_v[g, pl.ds(0, 16)]
                u1 = urow_v[g, pl.ds(16, 16)]
                i0 = irow_v[g, pl.ds(0, 16)]
                i1 = irow_v[g, pl.ds(16, 16)]
                s = (u0 + h0) * i0 + (u1 + h1) * i1 + bias_vec
                stage_v[e] = s
                return carry

            lax.fori_loop(0, _CHUNK, ex_body, 0)
            acc_a = plsc.load_gather(stage_v, [lane, jnp.zeros((16,), jnp.int32)])
            acc_b = plsc.load_gather(stage_v, [lane, jnp.full((16,), 1, jnp.int32)])
            for k in range(2, 16, 2):
                acc_a = acc_a + plsc.load_gather(
                    stage_v, [lane, jnp.full((16,), k, jnp.int32)])
                acc_b = acc_b + plsc.load_gather(
                    stage_v, [lane, jnp.full((16,), k + 1, jnp.int32)])
            out_v[pl.ds(c * _CHUNK, _CHUNK)] = acc_a + acc_b

            @pl.when(c + 2 < _NCHUNK)
            def _(b=b, c=c):
                start(c + 2, b)
        return carry

    lax.fori_loop(0, _NCHUNK // 2, outer, 0)
    pltpu.sync_copy(out_v, out_hbm.at[pl.ds(base, _PER_W)])


@jax.jit
def _sc_scores(user_ids, item_ids, hist_flat, user_table, item_table,
               hist_table, bias_vec):
    mesh = plsc.VectorSubcoreMesh(core_axis_name="c", subcore_axis_name="s")
    f = pl.kernel(
        _body,
        out_type=jax.ShapeDtypeStruct((_B,), jnp.float32),
        mesh=mesh,
        compiler_params=pltpu.CompilerParams(
            needs_layout_passes=False, use_tc_tiling_on_sc=False),
        scratch_types=[
            pltpu.VMEM((_PER_W,), jnp.int32),
            pltpu.VMEM((_PER_W,), jnp.int32),
            pltpu.VMEM((_PER_W, _D), jnp.float32),
            pltpu.VMEM((_PER_W, _D), jnp.float32),
            pltpu.VMEM((2, _ROWS), jnp.int32),
            pltpu.VMEM((2, _ROWS, _D), jnp.float32),
            pltpu.VMEM((16,), jnp.float32),
            pltpu.VMEM((_PER_W,), jnp.float32),
            pltpu.VMEM((_CHUNK, 16), jnp.float32),
            pltpu.SemaphoreType.DMA,
            pltpu.SemaphoreType.DMA,
            pltpu.SemaphoreType.DMA,
            pltpu.SemaphoreType.DMA,
        ],
    )
    return f(user_ids, item_ids, hist_flat, user_table, item_table,
             hist_table, bias_vec)


def kernel(user_ids, item_ids, hist_ids, user_table, item_table, hist_table,
           user_bias, item_bias):
    bias = _AVG + user_bias[0] + item_bias[0]
    bias_vec = jnp.zeros((16,), jnp.float32).at[0].set(bias)
    hist_flat = hist_ids.reshape(-1)
    return _sc_scores(user_ids, item_ids, hist_flat, user_table, item_table,
                      hist_table, bias_vec)


# speedup vs baseline: 1.8657x; 1.4695x over previous
import functools

import jax
import jax.numpy as jnp
from jax import lax
from jax.experimental import pallas as pl
from jax.experimental.pallas import tpu as pltpu
from jax.experimental.pallas import tpu_sc as plsc

_B, _L, _D = 16384, 50, 32
_AVG = 3.5
_NC, _NS = 2, 16
_NW = _NC * _NS
_PER_W = _B // _NW
_CHUNK = 16
_NCHUNK = _PER_W // _CHUNK
_ROWS = _CHUNK * _L
_INV_SQRT_L = float(1.0 / (_L ** 0.5))
_TILES = 1000000 // 8


def _full(v):
    return jnp.full((16,), v, jnp.int32)


def _ui_body(uid_hbm, iid_hbm, ut3_hbm, it3_hbm, bias_hbm,
             part1_hbm, irows_hbm,
             uid_v, iid_v, ubuf_v, ibuf_v, bias_v, p1_v, ir_v,
             sem_u0, sem_u1, sem_i0, sem_i1):
    wid = lax.axis_index("s") * _NC + lax.axis_index("c")
    base = wid * _PER_W
    usems = (sem_u0, sem_u1)
    isems = (sem_i0, sem_i1)
    lane = lax.iota(jnp.int32, 16)

    pltpu.sync_copy(bias_hbm, bias_v)
    bias_vec = bias_v[...]
    pltpu.sync_copy(uid_hbm.at[pl.ds(base, _PER_W)], uid_v)
    pltpu.sync_copy(iid_hbm.at[pl.ds(base, _PER_W)], iid_v)

    def start(c, b):
        uv = uid_v[pl.ds(c * _CHUNK, _CHUNK)]
        iv = iid_v[pl.ds(c * _CHUNK, _CHUNK)]
        for e in range(_CHUNK):
            utid = lax.shift_right_logical(uv[e], 3)
            itid = lax.shift_right_logical(iv[e], 3)
            pltpu.async_copy(ut3_hbm.at[utid], ubuf_v.at[b, e], usems[b])
            pltpu.async_copy(it3_hbm.at[itid], ibuf_v.at[b, e], isems[b])

    start(0, 0)
    start(1, 1)

    def outer(o, carry):
        for b in range(2):
            c = 2 * o + b
            for e in range(_CHUNK):
                pltpu.make_async_copy(ut3_hbm.at[0], ubuf_v.at[b, e],
                                      usems[b]).wait()
                pltpu.make_async_copy(it3_hbm.at[0], ibuf_v.at[b, e],
                                      isems[b]).wait()

            usub = jnp.bitwise_and(uid_v[pl.ds(c * _CHUNK, _CHUNK)], 7)
            isub = jnp.bitwise_and(iid_v[pl.ds(c * _CHUNK, _CHUNK)], 7)
            acc_a = bias_vec
            acc_b = jnp.zeros((16,), jnp.float32)
            for l in range(0, _D, 2):
                uv0 = plsc.load_gather(ubuf_v.at[b], [lane, usub, _full(l)])
                iv0 = plsc.load_gather(ibuf_v.at[b], [lane, isub, _full(l)])
                uv1 = plsc.load_gather(ubuf_v.at[b],
                                       [lane, usub, _full(l + 1)])
                iv1 = plsc.load_gather(ibuf_v.at[b],
                                       [lane, isub, _full(l + 1)])
                ir_v[pl.ds(l * _PER_W + c * _CHUNK, _CHUNK)] = iv0
                ir_v[pl.ds((l + 1) * _PER_W + c * _CHUNK, _CHUNK)] = iv1
                acc_a = acc_a + uv0 * iv0
                acc_b = acc_b + uv1 * iv1
            p1_v[pl.ds(c * _CHUNK, _CHUNK)] = acc_a + acc_b

            @pl.when(c + 2 < _NCHUNK)
            def _(b=b, c=c):
                start(c + 2, b)
        return carry

    lax.fori_loop(0, _NCHUNK // 2, outer, 0)
    pltpu.sync_copy(p1_v, part1_hbm.at[pl.ds(base, _PER_W)])
    pltpu.sync_copy(ir_v, irows_hbm.at[pl.ds(base * _D, _PER_W * _D)])


def _hist_body(hid_hbm, ht_hbm, irows_hbm,
               part2_hbm,
               hid_v, hrow_v, ir_v, p2_v, hpool_v,
               sem_h0, sem_h1):
    wid = lax.axis_index("s") * _NC + lax.axis_index("c")
    base = wid * _PER_W
    hbase = base * _L
    sems = (sem_h0, sem_h1)
    lane = lax.iota(jnp.int32, 16)

    pltpu.sync_copy(irows_hbm.at[pl.ds(base * _D, _PER_W * _D)], ir_v)

    def start(c, b):
        pltpu.sync_copy(hid_hbm.at[pl.ds(hbase + c * _ROWS, _ROWS)],
                        hid_v.at[b])
        pltpu.async_copy(ht_hbm.at[hid_v.at[b]], hrow_v.at[b], sems[b])

    start(0, 0)
    start(1, 1)

    def outer(o, carry):
        for b in range(2):
            c = 2 * o + b
            pltpu.make_async_copy(ht_hbm.at[hid_v.at[b]], hrow_v.at[b],
                                  sems[b]).wait()

            def ex_body(e, carry2, b=b):
                r0 = e * _L
                a0 = jnp.zeros((16,), jnp.float32)
                a1 = jnp.zeros((16,), jnp.float32)
                a2 = jnp.zeros((16,), jnp.float32)
                a3 = jnp.zeros((16,), jnp.float32)
                for l in range(0, _L, 2):
                    a0 = a0 + hrow_v[b, r0 + l, pl.ds(0, 16)]
                    a1 = a1 + hrow_v[b, r0 + l, pl.ds(16, 16)]
                    a2 = a2 + hrow_v[b, r0 + l + 1, pl.ds(0, 16)]
                    a3 = a3 + hrow_v[b, r0 + l + 1, pl.ds(16, 16)]
                hpool_v[e, pl.ds(0, 16)] = a0 + a2
                hpool_v[e, pl.ds(16, 16)] = a1 + a3
                return carry2

            lax.fori_loop(0, _CHUNK, ex_body, 0)

            acc_a = jnp.zeros((16,), jnp.float32)
            acc_b = jnp.zeros((16,), jnp.float32)
            for l in range(0, _D, 2):
                hv0 = plsc.load_gather(hpool_v, [lane, _full(l)])
                hv1 = plsc.load_gather(hpool_v, [lane, _full(l + 1)])
                iv0 = ir_v[pl.ds(l * _PER_W + c * _CHUNK, _CHUNK)]
                iv1 = ir_v[pl.ds((l + 1) * _PER_W + c * _CHUNK, _CHUNK)]
                acc_a = acc_a + hv0 * iv0
                acc_b = acc_b + hv1 * iv1
            p2_v[pl.ds(c * _CHUNK, _CHUNK)] = (acc_a + acc_b) * _INV_SQRT_L

            @pl.when(c + 2 < _NCHUNK)
            def _(b=b, c=c):
                start(c + 2, b)
        return carry

    lax.fori_loop(0, _NCHUNK // 2, outer, 0)
    pltpu.sync_copy(p2_v, part2_hbm.at[pl.ds(base, _PER_W)])


@jax.jit
def _svdpp(user_ids, item_ids, hist_flat, user_table, item_table,
           hist_table, bias_vec):
    mesh = plsc.VectorSubcoreMesh(core_axis_name="c", subcore_axis_name="s")

    ui = pl.kernel(
        _ui_body,
        out_type=[
            jax.ShapeDtypeStruct((_B,), jnp.float32),
            jax.ShapeDtypeStruct((_B * _D,), jnp.float32),
        ],
        mesh=mesh,
        compiler_params=pltpu.CompilerParams(
            needs_layout_passes=False, use_tc_tiling_on_sc=True),
        scratch_types=[
            pltpu.VMEM((_PER_W,), jnp.int32),
            pltpu.VMEM((_PER_W,), jnp.int32),
            pltpu.VMEM((2, _CHUNK, 8, _D), jnp.float32),
            pltpu.VMEM((2, _CHUNK, 8, _D), jnp.float32),
            pltpu.VMEM((16,), jnp.float32),
            pltpu.VMEM((_PER_W,), jnp.float32),
            pltpu.VMEM((_PER_W * _D,), jnp.float32),
            pltpu.SemaphoreType.DMA,
            pltpu.SemaphoreType.DMA,
            pltpu.SemaphoreType.DMA,
            pltpu.SemaphoreType.DMA,
        ],
    )
    ut3 = user_table.reshape(_TILES, 8, _D)
    it3 = item_table.reshape(_TILES, 8, _D)
    part1, irows = ui(user_ids, item_ids, ut3, it3, bias_vec)

    hist = pl.kernel(
        _hist_body,
        out_type=jax.ShapeDtypeStruct((_B,), jnp.float32),
        mesh=mesh,
        compiler_params=pltpu.CompilerParams(
            needs_layout_passes=False, use_tc_tiling_on_sc=False),
        scratch_types=[
            pltpu.VMEM((2, _ROWS), jnp.int32),
            pltpu.VMEM((2, _ROWS, _D), jnp.float32),
            pltpu.VMEM((_PER_W * _D,), jnp.float32),
            pltpu.VMEM((_PER_W,), jnp.float32),
            pltpu.VMEM((_CHUNK, _D), jnp.float32),
            pltpu.SemaphoreType.DMA,
            pltpu.SemaphoreType.DMA,
        ],
    )
    part2 = hist(hist_flat, hist_table, irows)
    return part1 + part2


def kernel(user_ids, item_ids, hist_ids, user_table, item_table, hist_table,
           user_bias, item_bias):
    bias = _AVG + user_bias[0] + item_bias[0]
    bias_vec = jnp.full((16,), bias, jnp.float32)
    hist_flat = hist_ids.reshape(-1)
    return _svdpp(user_ids, item_ids, hist_flat, user_table, item_table,
                  hist_table, bias_vec)
